# baseline (speedup 1.0000x reference)
import functools

import jax
import jax.numpy as jnp
from jax import lax
from jax.experimental import pallas as pl
from jax.experimental.pallas import tpu as pltpu
from jax.experimental.pallas import tpu_sc as plsc

N_NODES = 10000
N_EDGES = 160000
MAX_Z = 94
N_ORB = 37
EMB = 16
D_OUT = N_ORB * EMB

LANES = 16


def _gather_zj_sc(z, idx_j):
    info = plsc.get_sparse_core_info()
    nc, ns = info.num_cores, info.num_subcores
    nw = nc * ns
    epw = N_EDGES // nw
    n_iters = (epw + LANES - 1) // LANES
    buf = n_iters * LANES + LANES

    mesh = plsc.VectorSubcoreMesh(core_axis_name="c", subcore_axis_name="s")

    @functools.partial(
        pl.kernel,
        mesh=mesh,
        compiler_params=pltpu.CompilerParams(needs_layout_passes=False),
        out_type=jax.ShapeDtypeStruct((N_EDGES,), jnp.int32),
        scratch_types=[
            pltpu.VMEM((N_NODES,), jnp.int32),
            pltpu.VMEM((buf,), jnp.int32),
            pltpu.VMEM((buf,), jnp.int32),
        ],
    )
    def zj_kernel(z_hbm, idx_hbm, zj_hbm, z_v, idx_v, out_v):
        wid = lax.axis_index("s") * nc + lax.axis_index("c")
        base = wid * epw
        pltpu.sync_copy(z_hbm, z_v)
        pltpu.sync_copy(idx_hbm.at[pl.ds(base, epw)], idx_v.at[pl.ds(0, epw)])
        idx_v[pl.ds(epw, LANES)] = jnp.zeros((LANES,), jnp.int32)

        def body(i, carry):
            idx16 = idx_v[pl.ds(i * LANES, LANES)]
            out_v[pl.ds(i * LANES, LANES)] = plsc.load_gather(z_v, [idx16])
            return carry

        lax.fori_loop(0, n_iters, body, 0)
        pltpu.sync_copy(out_v.at[pl.ds(0, epw)], zj_hbm.at[pl.ds(base, epw)])

    return zj_kernel(z, idx_j)


_GR = 400
_NB = N_EDGES // (8 * _GR)


def _vexp_body(val_ref, vexp_ref):
    rep = (
        lax.broadcasted_iota(jnp.int32, (N_ORB, D_OUT), 1) // EMB
        == lax.broadcasted_iota(jnp.int32, (N_ORB, D_OUT), 0)
    ).astype(jnp.float32)
    vexp_ref[...] = jnp.dot(
        val_ref[...], rep, preferred_element_type=jnp.float32
    ).astype(jnp.bfloat16)


def _expand_tc_body(*refs):
    zj_refs, vexp_ref, out_ref = refs[:8], refs[8], refs[9]
    for p in range(8):
        zjb = jnp.broadcast_to(zj_refs[p][0], (MAX_Z, _GR))
        onehot_t = (zjb == lax.broadcasted_iota(jnp.int32, (MAX_Z, _GR), 0)).astype(
            jnp.bfloat16
        )
        m_p = lax.dot_general(
            onehot_t,
            vexp_ref[...],
            dimension_numbers=(((0,), (0,)), ((), ())),
            preferred_element_type=jnp.float32,
        )
        out_ref[:, D_OUT * p : D_OUT * (p + 1)] = m_p


def _expand_tc(zj, valence):
    vexp = pl.pallas_call(
        _vexp_body,
        out_shape=jax.ShapeDtypeStruct((MAX_Z, D_OUT), jnp.bfloat16),
    )(valence)
    out2 = pl.pallas_call(
        _expand_tc_body,
        grid=(_NB,),
        in_specs=[pl.BlockSpec((1, 1, _GR), lambda i: (i, 0, 0))] * 8
        + [pl.BlockSpec((MAX_Z, D_OUT), lambda i: (0, 0))],
        out_specs=pl.BlockSpec((_GR, 8 * D_OUT), lambda i: (i, 0)),
        out_shape=jax.ShapeDtypeStruct((N_EDGES // 8, 8 * D_OUT), jnp.float32),
    )(*[zj.reshape(_NB, _GR, 8)[:, :, p].reshape(_NB, 1, _GR) for p in range(8)],
      vexp)
    return out2.reshape(N_EDGES, N_ORB, EMB)


def kernel(z, idx_j, valence):
    zj = _gather_zj_sc(z, idx_j)
    return _expand_tc(zj, valence)


# speedup vs baseline: 1.0058x; 1.0058x over previous
import functools

import jax
import jax.numpy as jnp
from jax import lax
from jax.experimental import pallas as pl
from jax.experimental.pallas import tpu as pltpu
from jax.experimental.pallas import tpu_sc as plsc

N_NODES = 10000
N_EDGES = 160000
MAX_Z = 94
N_ORB = 37
EMB = 16
D_OUT = N_ORB * EMB

LANES = 16


def _gather_zj_sc(z, idx_j):
    info = plsc.get_sparse_core_info()
    nc, ns = info.num_cores, info.num_subcores
    nw = nc * ns
    epw = N_EDGES // nw
    n_iters = (epw + LANES - 1) // LANES
    buf = n_iters * LANES + LANES

    mesh = plsc.VectorSubcoreMesh(core_axis_name="c", subcore_axis_name="s")

    @functools.partial(
        pl.kernel,
        mesh=mesh,
        compiler_params=pltpu.CompilerParams(needs_layout_passes=False),
        out_type=jax.ShapeDtypeStruct((N_EDGES,), jnp.int32),
        scratch_types=[
            pltpu.VMEM((N_NODES,), jnp.int32),
            pltpu.VMEM((buf,), jnp.int32),
            pltpu.VMEM((buf,), jnp.int32),
        ],
    )
    def zj_kernel(z_hbm, idx_hbm, zj_hbm, z_v, idx_v, out_v):
        wid = lax.axis_index("s") * nc + lax.axis_index("c")
        base = wid * epw
        pltpu.sync_copy(z_hbm, z_v)
        pltpu.sync_copy(idx_hbm.at[pl.ds(base, epw)], idx_v.at[pl.ds(0, epw)])
        idx_v[pl.ds(epw, LANES)] = jnp.zeros((LANES,), jnp.int32)

        def body(i, carry):
            idx16 = idx_v[pl.ds(i * LANES, LANES)]
            out_v[pl.ds(i * LANES, LANES)] = plsc.load_gather(z_v, [idx16])
            return carry

        lax.fori_loop(0, n_iters, body, 0)
        pltpu.sync_copy(out_v.at[pl.ds(0, epw)], zj_hbm.at[pl.ds(base, epw)])

    return zj_kernel(z, idx_j)


_GR = 400
_NB = N_EDGES // (8 * _GR)


_D_ROW = 8 * D_OUT


def _segments():
    segs, t = [], 0
    while t < _D_ROW // 128:
        lo = 128 * t
        p_lo, p_hi = lo // D_OUT, (lo + 127) // D_OUT
        if p_lo == p_hi:
            t2 = t
            while t2 < _D_ROW // 128:
                lo2 = 128 * t2
                if lo2 // D_OUT == p_lo and (lo2 + 127) // D_OUT == p_lo:
                    t2 += 1
                else:
                    break
            segs.append((128 * t, 128 * t2, (p_lo,)))
            t = t2
        else:
            segs.append((lo, lo + 128, (p_lo, p_hi)))
            t += 1
    return segs


_SEGS = _segments()


def _vt_body(val_ref, vt_ref):
    rep8 = (
        (lax.broadcasted_iota(jnp.int32, (N_ORB, _D_ROW), 1) % D_OUT) // EMB
        == lax.broadcasted_iota(jnp.int32, (N_ORB, _D_ROW), 0)
    ).astype(jnp.float32)
    m1 = jnp.dot(val_ref[...], rep8, preferred_element_type=jnp.float32)
    n1 = lax.broadcasted_iota(jnp.int32, (MAX_Z, _D_ROW), 1)
    s = n1 // D_OUT
    p_lo = (n1 // 128 * 128) // D_OUT
    low = jnp.where(s == p_lo, m1, 0.0)
    high = jnp.where(s == p_lo + 1, m1, 0.0)
    vt_ref[...] = jnp.concatenate([low, high], axis=0).astype(jnp.bfloat16)


def _expand_tc_body(*refs):
    zj_refs, vt_ref, out_ref = refs[:8], refs[8], refs[9]
    ohs = []
    for p in range(8):
        zjb = jnp.broadcast_to(zj_refs[p][0], (MAX_Z, _GR))
        ohs.append(
            (zjb == lax.broadcasted_iota(jnp.int32, (MAX_Z, _GR), 0)).astype(
                jnp.bfloat16
            )
        )
    dn = (((0,), (0,)), ((), ()))
    for cs, ce, slots in _SEGS:
        if len(slots) == 1:
            lhs = ohs[slots[0]]
            rhs = vt_ref[0:MAX_Z, cs:ce]
        else:
            lhs = jnp.concatenate([ohs[slots[0]], ohs[slots[1]]], axis=0)
            rhs = vt_ref[:, cs:ce]
        out_ref[:, cs:ce] = lax.dot_general(
            lhs, rhs, dimension_numbers=dn, preferred_element_type=jnp.float32
        )


def _expand_tc(zj, valence):
    vt = pl.pallas_call(
        _vt_body,
        out_shape=jax.ShapeDtypeStruct((2 * MAX_Z, _D_ROW), jnp.bfloat16),
    )(valence)
    out2 = pl.pallas_call(
        _expand_tc_body,
        grid=(_NB,),
        in_specs=[pl.BlockSpec((1, 1, _GR), lambda i: (i, 0, 0))] * 8
        + [pl.BlockSpec((2 * MAX_Z, _D_ROW), lambda i: (0, 0))],
        out_specs=pl.BlockSpec((_GR, _D_ROW), lambda i: (i, 0)),
        out_shape=jax.ShapeDtypeStruct((N_EDGES // 8, _D_ROW), jnp.float32),
    )(*[zj.reshape(_NB, _GR, 8)[:, :, p].reshape(_NB, 1, _GR) for p in range(8)],
      vt)
    return out2.reshape(N_EDGES, N_ORB, EMB)


def kernel(z, idx_j, valence):
    zj = _gather_zj_sc(z, idx_j)
    return _expand_tc(zj, valence)


# speedup vs baseline: 25.5618x; 25.4144x over previous
import functools

import jax
import jax.numpy as jnp
from jax import lax
from jax.experimental import pallas as pl
from jax.experimental.pallas import tpu as pltpu
from jax.experimental.pallas import tpu_sc as plsc

N_NODES = 10000
N_EDGES = 160000
MAX_Z = 94
N_ORB = 37
EMB = 16
D_OUT = N_ORB * EMB

LANES = 16


def _gather_zj_sc(z, idx_j):
    info = plsc.get_sparse_core_info()
    nc, ns = info.num_cores, info.num_subcores
    nw = nc * ns
    epw = N_EDGES // nw
    n_iters = (epw + LANES - 1) // LANES
    buf = n_iters * LANES + LANES

    mesh = plsc.VectorSubcoreMesh(core_axis_name="c", subcore_axis_name="s")

    @functools.partial(
        pl.kernel,
        mesh=mesh,
        compiler_params=pltpu.CompilerParams(needs_layout_passes=False),
        out_type=jax.ShapeDtypeStruct((N_EDGES,), jnp.int32),
        scratch_types=[
            pltpu.VMEM((N_NODES,), jnp.int32),
            pltpu.VMEM((buf,), jnp.int32),
            pltpu.VMEM((buf,), jnp.int32),
        ],
    )
    def zj_kernel(z_hbm, idx_hbm, zj_hbm, z_v, idx_v, out_v):
        wid = lax.axis_index("s") * nc + lax.axis_index("c")
        base = wid * epw
        pltpu.sync_copy(z_hbm, z_v)
        pltpu.sync_copy(idx_hbm.at[pl.ds(base, epw)], idx_v.at[pl.ds(0, epw)])
        idx_v[pl.ds(epw, LANES)] = jnp.zeros((LANES,), jnp.int32)

        def body(i, carry):
            idx16 = idx_v[pl.ds(i * LANES, LANES)]
            out_v[pl.ds(i * LANES, LANES)] = plsc.load_gather(z_v, [idx16])
            return carry

        lax.fori_loop(0, n_iters, body, 0)
        pltpu.sync_copy(out_v.at[pl.ds(0, epw)], zj_hbm.at[pl.ds(base, epw)])

    return zj_kernel(z, idx_j)


_BE = 6400
_NB = N_EDGES // _BE


def _expand_tc_body(zj_ref, val_ref, out_ref):
    zjb = jnp.broadcast_to(zj_ref[0], (MAX_Z, _BE))
    onehot_t = (zjb == lax.broadcasted_iota(jnp.int32, (MAX_Z, _BE), 0)).astype(
        jnp.bfloat16
    )
    m = lax.dot_general(
        val_ref[...].astype(jnp.bfloat16),
        onehot_t,
        dimension_numbers=(((0,), (0,)), ((), ())),
        preferred_element_type=jnp.float32,
    )
    out_ref[...] = jnp.broadcast_to(m[:, None, :], (N_ORB, EMB, _BE))


def _expand_tc(zj, valence):
    out_t = pl.pallas_call(
        _expand_tc_body,
        grid=(_NB,),
        in_specs=[
            pl.BlockSpec((1, 1, _BE), lambda i: (i, 0, 0)),
            pl.BlockSpec((MAX_Z, N_ORB), lambda i: (0, 0)),
        ],
        out_specs=pl.BlockSpec((N_ORB, EMB, _BE), lambda i: (0, 0, i)),
        out_shape=jax.ShapeDtypeStruct((N_ORB, EMB, N_EDGES), jnp.float32),
    )(zj.reshape(_NB, 1, _BE), valence)
    return jnp.transpose(out_t, (2, 0, 1))


def kernel(z, idx_j, valence):
    zj = _gather_zj_sc(z, idx_j)
    return _expand_tc(zj, valence)
